# Initial kernel scaffold; baseline (speedup 1.0000x reference)
import jax
import jax.numpy as jnp
from jax.experimental import pallas as pl


def kernel(x, edge_index, edge_weight, Wl1, bl1, Wr1, pw1, Wl2, bl2, Wr2, pw2, Wl3, bl3, Wr3, pw3, W1, b1, W2, b2, W3, b3):
    raise NotImplementedError("write your pallas kernel here")


# speedup vs baseline: 60.3221x; 60.3221x over previous
import functools

import jax
import jax.numpy as jnp
from jax import lax
from jax.experimental import pallas as pl
from jax.experimental.pallas import tpu as pltpu
from jax.experimental.pallas import tpu_sc as plsc

_N = 10000
_E = 320000
_F = 128
_H = 16
_C = 10

_CHUNK = 128
_NC, _NS = 2, 16
_CORE_E = _E // _NC
_CORE_CHUNKS = _CORE_E // _CHUNK
_Q, _R = divmod(_CORE_CHUNKS, _NS)
_MAXC = _Q + 1
_EPAD = 320256
_RPT = _N // _NS
_GCH = 96
_GOUT = _GCH * _CHUNK
_SORT_N = 16384
_NEG_PAD = float("inf")

_f32 = jnp.float32


def _seg_sum_body(with_count, *refs):
    if with_count:
        (y_hbm, src_hbm, dst_hbm, s_out, c_out,
         y_sp, s_sp, c_sp, src_v, dst_v, rows_v, zb_v, ones_v, sem) = refs
    else:
        (y_hbm, src_hbm, dst_hbm, s_out,
         y_sp, s_sp, src_v, dst_v, rows_v, zb_v, sem) = refs
        c_out = c_sp = ones_v = None

    c = lax.axis_index("c")
    s = lax.axis_index("s")
    row0 = s * _RPT

    def zfill(i, _):
        zb_v[i] = jnp.zeros((_H,), _f32)
        return 0
    lax.fori_loop(0, _RPT, zfill, 0)
    pltpu.sync_copy(zb_v, s_sp.at[pl.ds(row0, _RPT)])
    if with_count:
        pltpu.sync_copy(zb_v, c_sp.at[pl.ds(row0, _RPT)])

        def ofill(i, _):
            ones_v[i] = jnp.ones((_H,), _f32)
            return 0
        lax.fori_loop(0, _CHUNK, ofill, 0)
    pltpu.sync_copy(y_hbm.at[pl.ds(row0, _RPT)], y_sp.at[pl.ds(row0, _RPT)])

    start_chunk = s * _Q + jnp.minimum(s, _R)
    nch = jnp.where(s < _R, _Q + 1, _Q)
    cb = c * _CORE_CHUNKS + start_chunk
    pltpu.sync_copy(src_hbm.at[pl.ds(cb, _MAXC)], src_v)
    pltpu.sync_copy(dst_hbm.at[pl.ds(cb, _MAXC)], dst_v)

    plsc.subcore_barrier()

    def step(j, _):
        pltpu.async_copy(y_sp.at[src_v.at[j]], rows_v, sem).wait()
        pltpu.sync_copy(rows_v, s_sp.at[dst_v.at[j]], add=True)
        if with_count:
            pltpu.sync_copy(ones_v, c_sp.at[dst_v.at[j]], add=True)
        return 0
    lax.fori_loop(0, nch, step, 0)

    plsc.subcore_barrier()

    out_row = c * _N + row0
    pltpu.sync_copy(s_sp.at[pl.ds(row0, _RPT)], s_out.at[pl.ds(out_row, _RPT)])
    if with_count:
        pltpu.sync_copy(c_sp.at[pl.ds(row0, _RPT)], c_out.at[pl.ds(out_row, _RPT)])


@functools.lru_cache(maxsize=None)
def _make_seg_sum(with_count):
    mesh = plsc.VectorSubcoreMesh(core_axis_name="c", subcore_axis_name="s", num_cores=_NC, num_subcores=_NS)
    outs = [jax.ShapeDtypeStruct((_NC * _N, _H), _f32)]
    scratch = [
        pltpu.VMEM_SHARED((_N, _H), _f32),
        pltpu.VMEM_SHARED((_N, _H), _f32),
    ]
    if with_count:
        outs.append(jax.ShapeDtypeStruct((_NC * _N, _H), _f32))
        scratch.append(pltpu.VMEM_SHARED((_N, _H), _f32))
    scratch += [
        pltpu.VMEM((_MAXC, _CHUNK), jnp.int32),
        pltpu.VMEM((_MAXC, _CHUNK), jnp.int32),
        pltpu.VMEM((_CHUNK, _H), _f32),
        pltpu.VMEM((_RPT, _H), _f32),
    ]
    if with_count:
        scratch.append(pltpu.VMEM((_CHUNK, _H), _f32))
    scratch.append(pltpu.SemaphoreType.DMA)
    return pl.kernel(
        functools.partial(_seg_sum_body, with_count),
        out_type=tuple(outs),
        mesh=mesh,
        scratch_types=tuple(scratch),
        compiler_params=pltpu.CompilerParams(use_tc_tiling_on_sc=False),
    )


def _seg_sum_cnt(y, src2, dst2):
    return _make_seg_sum(True)(y, src2, dst2)


def _seg_sum(y, src2, dst2):
    return _make_seg_sum(False)(y, src2, dst2)


def _gather_body(h_hbm, perm_hbm, out_hbm, idx_v, rows_v, sem):
    c = lax.axis_index("c")
    s = lax.axis_index("s")
    wid = s * _NC + c
    base = wid * 3
    pltpu.sync_copy(perm_hbm.at[pl.ds(base, 3)], idx_v)
    for j in range(3):
        pltpu.async_copy(h_hbm.at[idx_v.at[j]], rows_v, sem).wait()
        pltpu.sync_copy(rows_v, out_hbm.at[pl.ds((base + j) * _CHUNK, _CHUNK)])


@functools.lru_cache(maxsize=None)
def _make_gather_rows():
    return pl.kernel(
        _gather_body,
        out_type=jax.ShapeDtypeStruct((_GOUT, _H), _f32),
        mesh=plsc.VectorSubcoreMesh(core_axis_name="c", subcore_axis_name="s",
                                    num_cores=_NC, num_subcores=_NS),
        scratch_types=(
            pltpu.VMEM((3, _CHUNK), jnp.int32),
            pltpu.VMEM((_CHUNK, _H), _f32),
            pltpu.SemaphoreType.DMA,
        ),
        compiler_params=pltpu.CompilerParams(use_tc_tiling_on_sc=False),
    )


def _gather_rows(h, perm):
    return _make_gather_rows()(h, perm)


def _proj_body(x_ref, wl_ref, wr_ref, y_ref, z_ref):
    xv = x_ref[...]
    y_ref[...] = jnp.dot(xv, wl_ref[...], preferred_element_type=_f32)
    z_ref[...] = jnp.dot(xv, wr_ref[...], preferred_element_type=_f32)


def _proj(x, wl, wr):
    return pl.pallas_call(
        _proj_body,
        out_shape=(
            jax.ShapeDtypeStruct((x.shape[0], wl.shape[1]), _f32),
            jax.ShapeDtypeStruct((x.shape[0], wr.shape[1]), _f32),
        ),
    )(x, wl, wr)


def _combine1_body(sp_ref, cp_ref, z_ref, bl_ref, pw_ref, wl_ref, wr_ref,
                   y_ref, zn_ref, ns_ref, cc_ref):
    s = sp_ref[0:_N, :] + sp_ref[_N:2 * _N, :]
    cnt = cp_ref[0:_N, 0:1] + cp_ref[_N:2 * _N, 0:1]
    cc = jnp.maximum(cnt, 1.0)
    h = jax.nn.relu(s / cc + bl_ref[...] + z_ref[...])
    pw = pw_ref[...]
    score = jnp.sum(h * pw, axis=1, keepdims=True) / jnp.sqrt(jnp.sum(pw * pw))
    hg = h * jnp.tanh(score)
    y_ref[...] = jnp.dot(hg, wl_ref[...], preferred_element_type=_f32)
    zn_ref[...] = jnp.dot(hg, wr_ref[...], preferred_element_type=_f32)
    ns_ref[...] = -score
    cc_ref[...] = cc


def _combine2_body(sp_ref, cc_ref, z_ref, bl_ref, pw_ref, wl_ref, wr_ref,
                   y_ref, zn_ref, ns_ref):
    s = sp_ref[0:_N, :] + sp_ref[_N:2 * _N, :]
    h = jax.nn.relu(s / cc_ref[...] + bl_ref[...] + z_ref[...])
    pw = pw_ref[...]
    score = jnp.sum(h * pw, axis=1, keepdims=True) / jnp.sqrt(jnp.sum(pw * pw))
    hg = h * jnp.tanh(score)
    y_ref[...] = jnp.dot(hg, wl_ref[...], preferred_element_type=_f32)
    zn_ref[...] = jnp.dot(hg, wr_ref[...], preferred_element_type=_f32)
    ns_ref[...] = -score


def _combine3_body(sp_ref, cc_ref, z_ref, bl_ref, pw_ref, hg_ref, ns_ref):
    s = sp_ref[0:_N, :] + sp_ref[_N:2 * _N, :]
    h = jax.nn.relu(s / cc_ref[...] + bl_ref[...] + z_ref[...])
    pw = pw_ref[...]
    score = jnp.sum(h * pw, axis=1, keepdims=True) / jnp.sqrt(jnp.sum(pw * pw))
    hg_ref[...] = h * jnp.tanh(score)
    ns_ref[...] = -score


def _sort_body(k3_ref, k2_ref, k1_ref, perm_ref):
    k3 = k3_ref[...]
    k2 = k2_ref[...]
    k1 = k1_ref[...]
    rows, cols = k3.shape
    row = lax.broadcasted_iota(jnp.int32, (rows, cols), 0)
    col = lax.broadcasted_iota(jnp.int32, (rows, cols), 1)
    idx = row * cols + col

    def shifted(x, d, ax):
        n = x.shape[ax]
        d = d % n
        if ax == 0:
            return jnp.concatenate([x[d:, :], x[:d, :]], axis=0)
        return jnp.concatenate([x[:, d:], x[:, :d]], axis=1)

    for ke in range(1, 15):
        big = 1 << ke
        for je in range(ke - 1, -1, -1):
            d = 1 << je
            if d >= cols:
                ax, sh = 0, d // cols
            else:
                ax, sh = 1, d
            lower = (idx & d) == 0
            asc = (idx & big) == 0
            p3 = jnp.where(lower, shifted(k3, sh, ax), shifted(k3, -sh, ax))
            p2 = jnp.where(lower, shifted(k2, sh, ax), shifted(k2, -sh, ax))
            p1 = jnp.where(lower, shifted(k1, sh, ax), shifted(k1, -sh, ax))
            pi = jnp.where(lower, shifted(idx, sh, ax), shifted(idx, -sh, ax))
            lt = (k3 < p3) | ((k3 == p3) & (
                (k2 < p2) | ((k2 == p2) & (
                    (k1 < p1) | ((k1 == p1) & (idx < pi))))))
            take = lt == (lower == asc)
            k3 = jnp.where(take, k3, p3)
            k2 = jnp.where(take, k2, p2)
            k1 = jnp.where(take, k1, p1)
            idx = jnp.where(take, idx, pi)

    perm_ref[...] = jnp.minimum(idx, _N - 1)


def _sort(k3, k2, k1):
    return pl.pallas_call(
        _sort_body,
        out_shape=jax.ShapeDtypeStruct(k3.shape, jnp.int32),
    )(k3, k2, k1)


def _mlp_body(h_ref, w1_ref, b1_ref, w2_ref, b2_ref, w3_ref, b3_ref, o_ref):
    m = jax.nn.relu(
        jnp.dot(h_ref[...], w1_ref[...], preferred_element_type=_f32)
        + b1_ref[...])
    m = jax.nn.relu(
        jnp.dot(m, w2_ref[...], preferred_element_type=_f32) + b2_ref[...])
    lg = jnp.dot(m, w3_ref[...], preferred_element_type=_f32) + b3_ref[...]
    mx = jnp.max(lg, axis=1, keepdims=True)
    sh = lg - mx
    o_ref[...] = sh - jnp.log(jnp.sum(jnp.exp(sh), axis=1, keepdims=True))


def _mlp(h, w1, b1, w2, b2, w3, b3):
    return pl.pallas_call(
        _mlp_body,
        out_shape=jax.ShapeDtypeStruct((h.shape[0], _C), _f32),
    )(h, w1, b1, w2, b2, w3, b3)


def kernel(x, edge_index, edge_weight, Wl1, bl1, Wr1, pw1, Wl2, bl2, Wr2, pw2,
           Wl3, bl3, Wr3, pw3, W1, b1, W2, b2, W3, b3):
    del edge_weight

    src = edge_index[0]
    dst = edge_index[1]
    pad = _EPAD - _E
    src2 = jnp.pad(src, (0, pad)).reshape(_EPAD // _CHUNK, _CHUNK)
    dst2 = jnp.pad(dst, (0, pad)).reshape(_EPAD // _CHUNK, _CHUNK)

    bl1r = bl1.reshape(1, _H)
    bl2r = bl2.reshape(1, _H)
    bl3r = bl3.reshape(1, _H)
    pw1r = pw1.reshape(1, _H)
    pw2r = pw2.reshape(1, _H)
    pw3r = pw3.reshape(1, _H)

    y1, z1 = _proj(x, Wl1, Wr1)
    s1p, c1p = _seg_sum_cnt(y1, src2, dst2)
    y2, z2, ns1, cc = pl.pallas_call(
        _combine1_body,
        out_shape=(
            jax.ShapeDtypeStruct((_N, _H), _f32),
            jax.ShapeDtypeStruct((_N, _H), _f32),
            jax.ShapeDtypeStruct((_N, 1), _f32),
            jax.ShapeDtypeStruct((_N, 1), _f32),
        ),
    )(s1p, c1p, z1, bl1r, pw1r, Wl2, Wr2)

    s2p, = _seg_sum(y2, src2, dst2)
    y3, z3, ns2 = pl.pallas_call(
        _combine2_body,
        out_shape=(
            jax.ShapeDtypeStruct((_N, _H), _f32),
            jax.ShapeDtypeStruct((_N, _H), _f32),
            jax.ShapeDtypeStruct((_N, 1), _f32),
        ),
    )(s2p, cc, z2, bl2r, pw2r, Wl3, Wr3)

    s3p, = _seg_sum(y3, src2, dst2)
    hg3, ns3 = pl.pallas_call(
        _combine3_body,
        out_shape=(
            jax.ShapeDtypeStruct((_N, _H), _f32),
            jax.ShapeDtypeStruct((_N, 1), _f32),
        ),
    )(s3p, cc, z3, bl3r, pw3r)

    def pad_key(ns):
        return jnp.pad(ns[:, 0], (0, _SORT_N - _N),
                       constant_values=_NEG_PAD).reshape(128, 128)

    perm2d = _sort(pad_key(ns3), pad_key(ns2), pad_key(ns1))
    perm = perm2d.reshape(_SORT_N)[:_GOUT].reshape(_GCH, _CHUNK)

    hperm = _gather_rows(hg3, perm)

    return _mlp(hperm[:_N], W1, b1.reshape(1, _H), W2, b2.reshape(1, 8),
                W3, b3.reshape(1, _C))


# speedup vs baseline: 69.8073x; 1.1572x over previous
import functools

import jax
import jax.numpy as jnp
from jax import lax
from jax.experimental import pallas as pl
from jax.experimental.pallas import tpu as pltpu
from jax.experimental.pallas import tpu_sc as plsc

_N = 10000
_E = 320000
_F = 128
_H = 16
_C = 10

_CHUNK = 128
_NC, _NS = 2, 16
_CORE_E = _E // _NC
_CORE_CHUNKS = _CORE_E // _CHUNK
_Q = 78
_MAXC = 80
_RPT = _N // _NS
_GCH = 96
_GOUT = _GCH * _CHUNK
_SORT_N = 16384
_NEG_PAD = float("inf")

_f32 = jnp.float32


def _seg_sum_body(with_count, *refs):
    if with_count:
        (y_hbm, ei_hbm, s_out, c_out,
         y_sp, s_sp, c_sp, src_v, dst_v, rows_a, rows_b, zb_v, ones_v,
         sem_a, sem_b) = refs
    else:
        (y_hbm, ei_hbm, s_out,
         y_sp, s_sp, src_v, dst_v, rows_a, rows_b, zb_v,
         sem_a, sem_b) = refs
        c_out = c_sp = ones_v = None

    c = lax.axis_index("c")
    s = lax.axis_index("s")
    row0 = s * _RPT

    def zfill(i, _):
        zb_v[i] = jnp.zeros((_H,), _f32)
        return 0
    lax.fori_loop(0, _RPT, zfill, 0)
    pltpu.sync_copy(zb_v, s_sp.at[pl.ds(row0, _RPT)])
    if with_count:
        pltpu.sync_copy(zb_v, c_sp.at[pl.ds(row0, _RPT)])

        def ofill(i, _):
            ones_v[i] = jnp.ones((_H,), _f32)
            return 0
        lax.fori_loop(0, _CHUNK, ofill, 0)
    pltpu.sync_copy(y_hbm.at[pl.ds(row0, _RPT)], y_sp.at[pl.ds(row0, _RPT)])

    nch = jnp.where(s < _NS - 1, _Q, _MAXC)
    cb = c * _CORE_CHUNKS + s * _Q
    pltpu.sync_copy(ei_hbm.at[0, pl.ds(cb, _MAXC)], src_v)
    pltpu.sync_copy(ei_hbm.at[1, pl.ds(cb, _MAXC)], dst_v)

    plsc.subcore_barrier()

    def gather(j, buf, sem):
        return pltpu.async_copy(y_sp.at[src_v.at[j]], buf, sem)

    def drain(j, buf, sem):
        pltpu.make_async_copy(y_sp.at[src_v.at[j]], buf, sem).wait()

    def scatter(j, buf):
        pltpu.sync_copy(buf, s_sp.at[dst_v.at[j]], add=True)
        if with_count:
            pltpu.sync_copy(ones_v, c_sp.at[dst_v.at[j]], add=True)

    gather(0, rows_a, sem_a)
    gather(1, rows_b, sem_b)

    def pair(jj, _):
        j0 = 2 * jj
        drain(j0, rows_a, sem_a)
        scatter(j0, rows_a)
        gather(j0 + 2, rows_a, sem_a)
        drain(j0 + 1, rows_b, sem_b)
        scatter(j0 + 1, rows_b)
        gather(j0 + 3, rows_b, sem_b)
        return 0
    lax.fori_loop(0, nch // 2 - 1, pair, 0)

    jl = nch - 2
    drain(jl, rows_a, sem_a)
    scatter(jl, rows_a)
    drain(jl + 1, rows_b, sem_b)
    scatter(jl + 1, rows_b)

    plsc.subcore_barrier()

    out_row = c * _N + row0
    pltpu.sync_copy(s_sp.at[pl.ds(row0, _RPT)], s_out.at[pl.ds(out_row, _RPT)])
    if with_count:
        pltpu.sync_copy(c_sp.at[pl.ds(row0, _RPT)], c_out.at[pl.ds(out_row, _RPT)])


@functools.lru_cache(maxsize=None)
def _make_seg_sum(with_count):
    mesh = plsc.VectorSubcoreMesh(core_axis_name="c", subcore_axis_name="s", num_cores=_NC, num_subcores=_NS)
    outs = [jax.ShapeDtypeStruct((_NC * _N, _H), _f32)]
    scratch = [
        pltpu.VMEM_SHARED((_N, _H), _f32),
        pltpu.VMEM_SHARED((_N, _H), _f32),
    ]
    if with_count:
        outs.append(jax.ShapeDtypeStruct((_NC * _N, _H), _f32))
        scratch.append(pltpu.VMEM_SHARED((_N, _H), _f32))
    scratch += [
        pltpu.VMEM((_MAXC, _CHUNK), jnp.int32),
        pltpu.VMEM((_MAXC, _CHUNK), jnp.int32),
        pltpu.VMEM((_CHUNK, _H), _f32),
        pltpu.VMEM((_CHUNK, _H), _f32),
        pltpu.VMEM((_RPT, _H), _f32),
    ]
    if with_count:
        scratch.append(pltpu.VMEM((_CHUNK, _H), _f32))
    scratch.append(pltpu.SemaphoreType.DMA)
    scratch.append(pltpu.SemaphoreType.DMA)
    return pl.kernel(
        functools.partial(_seg_sum_body, with_count),
        out_type=tuple(outs),
        mesh=mesh,
        scratch_types=tuple(scratch),
        compiler_params=pltpu.CompilerParams(use_tc_tiling_on_sc=False),
    )


def _seg_sum_cnt(y, ei3):
    return _make_seg_sum(True)(y, ei3)


def _seg_sum(y, ei3):
    return _make_seg_sum(False)(y, ei3)


def _gather_body(h_hbm, perm_hbm, out_hbm, idx_v, rows_v, sem0, sem1, sem2):
    c = lax.axis_index("c")
    s = lax.axis_index("s")
    wid = s * _NC + c
    base = wid * 3
    sems = (sem0, sem1, sem2)
    pltpu.sync_copy(perm_hbm.at[pl.ds(base, 3)], idx_v)
    cps = [pltpu.async_copy(h_hbm.at[idx_v.at[j]], rows_v.at[j], sems[j])
           for j in range(3)]
    for j in range(3):
        cps[j].wait()
        pltpu.sync_copy(rows_v.at[j], out_hbm.at[pl.ds((base + j) * _CHUNK, _CHUNK)])


@functools.lru_cache(maxsize=None)
def _make_gather_rows():
    return pl.kernel(
        _gather_body,
        out_type=jax.ShapeDtypeStruct((_GOUT, _H), _f32),
        mesh=plsc.VectorSubcoreMesh(core_axis_name="c", subcore_axis_name="s",
                                    num_cores=_NC, num_subcores=_NS),
        scratch_types=(
            pltpu.VMEM((3, _CHUNK), jnp.int32),
            pltpu.VMEM((3, _CHUNK, _H), _f32),
            pltpu.SemaphoreType.DMA,
            pltpu.SemaphoreType.DMA,
            pltpu.SemaphoreType.DMA,
        ),
        compiler_params=pltpu.CompilerParams(use_tc_tiling_on_sc=False),
    )


def _gather_rows(h, perm):
    return _make_gather_rows()(h, perm)


def _proj_body(x_ref, wl_ref, wr_ref, y_ref, z_ref):
    xv = x_ref[...]
    y_ref[...] = jnp.dot(xv, wl_ref[...], preferred_element_type=_f32)
    z_ref[...] = jnp.dot(xv, wr_ref[...], preferred_element_type=_f32)


def _proj(x, wl, wr):
    return pl.pallas_call(
        _proj_body,
        out_shape=(
            jax.ShapeDtypeStruct((x.shape[0], wl.shape[1]), _f32),
            jax.ShapeDtypeStruct((x.shape[0], wr.shape[1]), _f32),
        ),
    )(x, wl, wr)


def _combine1_body(sp_ref, cp_ref, z_ref, bl_ref, pw_ref, wl_ref, wr_ref,
                   y_ref, zn_ref, ns_ref, cc_ref):
    s = sp_ref[0:_N, :] + sp_ref[_N:2 * _N, :]
    cnt = cp_ref[0:_N, 0:1] + cp_ref[_N:2 * _N, 0:1]
    cc = jnp.maximum(cnt, 1.0)
    h = jax.nn.relu(s / cc + bl_ref[...] + z_ref[...])
    pw = pw_ref[...]
    score = jnp.sum(h * pw, axis=1, keepdims=True) / jnp.sqrt(jnp.sum(pw * pw))
    hg = h * jnp.tanh(score)
    y_ref[...] = jnp.dot(hg, wl_ref[...], preferred_element_type=_f32)
    zn_ref[...] = jnp.dot(hg, wr_ref[...], preferred_element_type=_f32)
    ns_ref[...] = -score
    cc_ref[...] = cc


def _combine2_body(sp_ref, cc_ref, z_ref, bl_ref, pw_ref, wl_ref, wr_ref,
                   y_ref, zn_ref, ns_ref):
    s = sp_ref[0:_N, :] + sp_ref[_N:2 * _N, :]
    h = jax.nn.relu(s / cc_ref[...] + bl_ref[...] + z_ref[...])
    pw = pw_ref[...]
    score = jnp.sum(h * pw, axis=1, keepdims=True) / jnp.sqrt(jnp.sum(pw * pw))
    hg = h * jnp.tanh(score)
    y_ref[...] = jnp.dot(hg, wl_ref[...], preferred_element_type=_f32)
    zn_ref[...] = jnp.dot(hg, wr_ref[...], preferred_element_type=_f32)
    ns_ref[...] = -score


def _combine3_body(sp_ref, cc_ref, z_ref, bl_ref, pw_ref, hg_ref, ns_ref):
    s = sp_ref[0:_N, :] + sp_ref[_N:2 * _N, :]
    h = jax.nn.relu(s / cc_ref[...] + bl_ref[...] + z_ref[...])
    pw = pw_ref[...]
    score = jnp.sum(h * pw, axis=1, keepdims=True) / jnp.sqrt(jnp.sum(pw * pw))
    hg_ref[...] = h * jnp.tanh(score)
    ns_ref[...] = -score


def _sort_body(k3_ref, k2_ref, k1_ref, perm_ref):
    k3 = k3_ref[...]
    k2 = k2_ref[...]
    k1 = k1_ref[...]
    rows, cols = k3.shape
    row = lax.broadcasted_iota(jnp.int32, (rows, cols), 0)
    col = lax.broadcasted_iota(jnp.int32, (rows, cols), 1)
    idx = row * cols + col

    def shifted(x, d, ax):
        n = x.shape[ax]
        d = d % n
        if ax == 0:
            return jnp.concatenate([x[d:, :], x[:d, :]], axis=0)
        return jnp.concatenate([x[:, d:], x[:, :d]], axis=1)

    for ke in range(1, 15):
        big = 1 << ke
        for je in range(ke - 1, -1, -1):
            d = 1 << je
            if d >= cols:
                ax, sh = 0, d // cols
            else:
                ax, sh = 1, d
            lower = (idx & d) == 0
            asc = (idx & big) == 0
            p3 = jnp.where(lower, shifted(k3, sh, ax), shifted(k3, -sh, ax))
            p2 = jnp.where(lower, shifted(k2, sh, ax), shifted(k2, -sh, ax))
            p1 = jnp.where(lower, shifted(k1, sh, ax), shifted(k1, -sh, ax))
            pi = jnp.where(lower, shifted(idx, sh, ax), shifted(idx, -sh, ax))
            lt = (k3 < p3) | ((k3 == p3) & (
                (k2 < p2) | ((k2 == p2) & (
                    (k1 < p1) | ((k1 == p1) & (idx < pi))))))
            take = lt == (lower == asc)
            k3 = jnp.where(take, k3, p3)
            k2 = jnp.where(take, k2, p2)
            k1 = jnp.where(take, k1, p1)
            idx = jnp.where(take, idx, pi)

    perm_ref[...] = jnp.minimum(idx, _N - 1)


def _sort(k3, k2, k1):
    return pl.pallas_call(
        _sort_body,
        out_shape=jax.ShapeDtypeStruct(k3.shape, jnp.int32),
    )(k3, k2, k1)


def _mlp_body(h_ref, w1_ref, b1_ref, w2_ref, b2_ref, w3_ref, b3_ref, o_ref):
    m = jax.nn.relu(
        jnp.dot(h_ref[...], w1_ref[...], preferred_element_type=_f32)
        + b1_ref[...])
    m = jax.nn.relu(
        jnp.dot(m, w2_ref[...], preferred_element_type=_f32) + b2_ref[...])
    lg = jnp.dot(m, w3_ref[...], preferred_element_type=_f32) + b3_ref[...]
    mx = jnp.max(lg, axis=1, keepdims=True)
    sh = lg - mx
    o_ref[...] = sh - jnp.log(jnp.sum(jnp.exp(sh), axis=1, keepdims=True))


def _mlp(h, w1, b1, w2, b2, w3, b3):
    return pl.pallas_call(
        _mlp_body,
        out_shape=jax.ShapeDtypeStruct((h.shape[0], _C), _f32),
    )(h, w1, b1, w2, b2, w3, b3)


def kernel(x, edge_index, edge_weight, Wl1, bl1, Wr1, pw1, Wl2, bl2, Wr2, pw2,
           Wl3, bl3, Wr3, pw3, W1, b1, W2, b2, W3, b3):
    del edge_weight

    ei3 = edge_index.reshape(2, _E // _CHUNK, _CHUNK)

    bl1r = bl1.reshape(1, _H)
    bl2r = bl2.reshape(1, _H)
    bl3r = bl3.reshape(1, _H)
    pw1r = pw1.reshape(1, _H)
    pw2r = pw2.reshape(1, _H)
    pw3r = pw3.reshape(1, _H)

    y1, z1 = _proj(x, Wl1, Wr1)
    s1p, c1p = _seg_sum_cnt(y1, ei3)
    y2, z2, ns1, cc = pl.pallas_call(
        _combine1_body,
        out_shape=(
            jax.ShapeDtypeStruct((_N, _H), _f32),
            jax.ShapeDtypeStruct((_N, _H), _f32),
            jax.ShapeDtypeStruct((_N, 1), _f32),
            jax.ShapeDtypeStruct((_N, 1), _f32),
        ),
    )(s1p, c1p, z1, bl1r, pw1r, Wl2, Wr2)

    s2p, = _seg_sum(y2, ei3)
    y3, z3, ns2 = pl.pallas_call(
        _combine2_body,
        out_shape=(
            jax.ShapeDtypeStruct((_N, _H), _f32),
            jax.ShapeDtypeStruct((_N, _H), _f32),
            jax.ShapeDtypeStruct((_N, 1), _f32),
        ),
    )(s2p, cc, z2, bl2r, pw2r, Wl3, Wr3)

    s3p, = _seg_sum(y3, ei3)
    hg3, ns3 = pl.pallas_call(
        _combine3_body,
        out_shape=(
            jax.ShapeDtypeStruct((_N, _H), _f32),
            jax.ShapeDtypeStruct((_N, 1), _f32),
        ),
    )(s3p, cc, z3, bl3r, pw3r)

    def pad_key(ns):
        return jnp.pad(ns[:, 0], (0, _SORT_N - _N),
                       constant_values=_NEG_PAD).reshape(128, 128)

    perm2d = _sort(pad_key(ns3), pad_key(ns2), pad_key(ns1))
    perm = perm2d.reshape(_SORT_N)[:_GOUT].reshape(_GCH, _CHUNK)

    hperm = _gather_rows(hg3, perm)

    return _mlp(hperm[:_N], W1, b1.reshape(1, _H), W2, b2.reshape(1, 8),
                W3, b3.reshape(1, _C))


# speedup vs baseline: 108.6578x; 1.5565x over previous
import functools

import jax
import jax.numpy as jnp
from jax import lax
from jax.experimental import pallas as pl
from jax.experimental.pallas import tpu as pltpu
from jax.experimental.pallas import tpu_sc as plsc

_N = 10000
_E = 320000
_F = 128
_H = 16
_C = 10

_CHUNK = 128
_NC, _NS = 2, 16
_CORE_E = _E // _NC
_CORE_CHUNKS = _CORE_E // _CHUNK
_Q = 78
_MAXC = 80
_NP = 10240
_PR = _NP // 8
_PRT = _PR // _NS
_RPT = _NP // _NS
_GCH = 96
_GOUT = _GCH * _CHUNK
_SORT_N = 16384
_NEG_PAD = float("inf")

_f32 = jnp.float32


def _seg_sum_body(with_count, *refs):
    if with_count:
        (y_hbm, ei_hbm, s_out, c_out,
         y_sp, s_sp, c_sp, src_v, dst_v, rows_a, rows_b, zb_v, b128_v, ones_v,
         sem_a, sem_b) = refs
    else:
        (y_hbm, ei_hbm, s_out,
         y_sp, s_sp, src_v, dst_v, rows_a, rows_b, zb_v, b128_v,
         sem_a, sem_b) = refs
        c_out = c_sp = ones_v = None

    c = lax.axis_index("c")
    s = lax.axis_index("s")
    row0 = s * _RPT
    prow0 = s * _PRT

    def zfill(i, _):
        zb_v[i] = jnp.zeros((_H,), _f32)
        return 0
    lax.fori_loop(0, _RPT, zfill, 0)
    pltpu.sync_copy(zb_v, s_sp.at[pl.ds(row0, _RPT)])
    if with_count:
        pltpu.sync_copy(zb_v, c_sp.at[pl.ds(row0, _RPT)])

        def ofill(i, _):
            ones_v[i] = jnp.ones((_H,), _f32)
            return 0
        lax.fori_loop(0, _CHUNK, ofill, 0)

    pltpu.sync_copy(y_hbm.at[pl.ds(prow0, _PRT)], b128_v)

    def unpackrow(r, _):
        for k in range(8):
            zb_v[r * 8 + k] = b128_v[r, pl.ds(k * _H, _H)]
        return 0
    lax.fori_loop(0, _PRT, unpackrow, 0)
    pltpu.sync_copy(zb_v, y_sp.at[pl.ds(row0, _RPT)])

    nch = jnp.where(s < _NS - 1, _Q, _MAXC)
    cb = c * _CORE_CHUNKS + s * _Q
    pltpu.sync_copy(ei_hbm.at[0, pl.ds(cb, _MAXC)], src_v)
    pltpu.sync_copy(ei_hbm.at[1, pl.ds(cb, _MAXC)], dst_v)

    plsc.subcore_barrier()

    def gather(j, buf, sem):
        return pltpu.async_copy(y_sp.at[src_v.at[j]], buf, sem)

    def drain(j, buf, sem):
        pltpu.make_async_copy(y_sp.at[src_v.at[j]], buf, sem).wait()

    def scatter(j, buf):
        pltpu.sync_copy(buf, s_sp.at[dst_v.at[j]], add=True)
        if with_count:
            pltpu.sync_copy(ones_v, c_sp.at[dst_v.at[j]], add=True)

    gather(0, rows_a, sem_a)
    gather(1, rows_b, sem_b)

    def pair(jj, _):
        j0 = 2 * jj
        drain(j0, rows_a, sem_a)
        scatter(j0, rows_a)
        gather(j0 + 2, rows_a, sem_a)
        drain(j0 + 1, rows_b, sem_b)
        scatter(j0 + 1, rows_b)
        gather(j0 + 3, rows_b, sem_b)
        return 0
    lax.fori_loop(0, nch // 2 - 1, pair, 0)

    jl = nch - 2
    drain(jl, rows_a, sem_a)
    scatter(jl, rows_a)
    drain(jl + 1, rows_b, sem_b)
    scatter(jl + 1, rows_b)

    plsc.subcore_barrier()

    def packrow(r, _):
        for k in range(8):
            b128_v[r, pl.ds(k * _H, _H)] = zb_v[r * 8 + k]
        return 0

    out_prow = c * _PR + prow0
    pltpu.sync_copy(s_sp.at[pl.ds(row0, _RPT)], zb_v)
    lax.fori_loop(0, _PRT, packrow, 0)
    pltpu.sync_copy(b128_v, s_out.at[pl.ds(out_prow, _PRT)])
    if with_count:
        pltpu.sync_copy(c_sp.at[pl.ds(row0, _RPT)], zb_v)
        lax.fori_loop(0, _PRT, packrow, 0)
        pltpu.sync_copy(b128_v, c_out.at[pl.ds(out_prow, _PRT)])


@functools.lru_cache(maxsize=None)
def _make_seg_sum(with_count):
    mesh = plsc.VectorSubcoreMesh(core_axis_name="c", subcore_axis_name="s", num_cores=_NC, num_subcores=_NS)
    outs = [jax.ShapeDtypeStruct((_NC * _PR, 8 * _H), _f32)]
    scratch = [
        pltpu.VMEM_SHARED((_NP, _H), _f32),
        pltpu.VMEM_SHARED((_NP, _H), _f32),
    ]
    if with_count:
        outs.append(jax.ShapeDtypeStruct((_NC * _PR, 8 * _H), _f32))
        scratch.append(pltpu.VMEM_SHARED((_NP, _H), _f32))
    scratch += [
        pltpu.VMEM((_MAXC, _CHUNK), jnp.int32),
        pltpu.VMEM((_MAXC, _CHUNK), jnp.int32),
        pltpu.VMEM((_CHUNK, _H), _f32),
        pltpu.VMEM((_CHUNK, _H), _f32),
        pltpu.VMEM((_RPT, _H), _f32),
        pltpu.VMEM((_PRT, 8 * _H), _f32),
    ]
    if with_count:
        scratch.append(pltpu.VMEM((_CHUNK, _H), _f32))
    scratch.append(pltpu.SemaphoreType.DMA)
    scratch.append(pltpu.SemaphoreType.DMA)
    return pl.kernel(
        functools.partial(_seg_sum_body, with_count),
        out_type=tuple(outs),
        mesh=mesh,
        scratch_types=tuple(scratch),
        compiler_params=pltpu.CompilerParams(use_tc_tiling_on_sc=False),
    )


def _seg_sum_cnt(y, ei3):
    return _make_seg_sum(True)(y, ei3)


def _seg_sum(y, ei3):
    return _make_seg_sum(False)(y, ei3)


def _gather_body(h_hbm, perm_hbm, out_hbm, h_sp, idx_v, rows_v, b16_v, b128_v,
                 sem0, sem1, sem2):
    c = lax.axis_index("c")
    s = lax.axis_index("s")
    wid = s * _NC + c
    base = wid * 3

    pltpu.sync_copy(h_hbm.at[pl.ds(s * _PRT, _PRT)], b128_v)

    def unpackrow(r, _):
        for k in range(8):
            b16_v[r * 8 + k] = b128_v[r, pl.ds(k * _H, _H)]
        return 0
    lax.fori_loop(0, _PRT, unpackrow, 0)
    pltpu.sync_copy(b16_v, h_sp.at[pl.ds(s * _RPT, _RPT)])

    sems = (sem0, sem1, sem2)
    pltpu.sync_copy(perm_hbm.at[pl.ds(base, 3)], idx_v)
    plsc.subcore_barrier()
    cps = [pltpu.async_copy(h_sp.at[idx_v.at[j]], rows_v.at[j], sems[j])
           for j in range(3)]
    for j in range(3):
        cps[j].wait()
        pltpu.sync_copy(rows_v.at[j], out_hbm.at[pl.ds((base + j) * _CHUNK, _CHUNK)])


@functools.lru_cache(maxsize=None)
def _make_gather_rows():
    return pl.kernel(
        _gather_body,
        out_type=jax.ShapeDtypeStruct((_GOUT, _H), _f32),
        mesh=plsc.VectorSubcoreMesh(core_axis_name="c", subcore_axis_name="s",
                                    num_cores=_NC, num_subcores=_NS),
        scratch_types=(
            pltpu.VMEM_SHARED((_NP, _H), _f32),
            pltpu.VMEM((3, _CHUNK), jnp.int32),
            pltpu.VMEM((3, _CHUNK, _H), _f32),
            pltpu.VMEM((_RPT, _H), _f32),
            pltpu.VMEM((_PRT, 8 * _H), _f32),
            pltpu.SemaphoreType.DMA,
            pltpu.SemaphoreType.DMA,
            pltpu.SemaphoreType.DMA,
        ),
        compiler_params=pltpu.CompilerParams(use_tc_tiling_on_sc=False),
    )


def _gather_rows(h, perm):
    return _make_gather_rows()(h, perm)


def _proj_body(x_ref, wl_ref, wr_ref, y_ref, z_ref):
    xv = x_ref[...]
    zpad = jnp.zeros((_PR - _N // 8, 8 * _H), _f32)
    y = jnp.dot(xv, wl_ref[...], preferred_element_type=_f32)
    z = jnp.dot(xv, wr_ref[...], preferred_element_type=_f32)
    y_ref[...] = jnp.concatenate([y, zpad], axis=0)
    z_ref[...] = jnp.concatenate([z, zpad], axis=0)


def _proj(xp, wlp, wrp):
    return pl.pallas_call(
        _proj_body,
        out_shape=(
            jax.ShapeDtypeStruct((_PR, 8 * _H), _f32),
            jax.ShapeDtypeStruct((_PR, 8 * _H), _f32),
        ),
    )(xp, wlp, wrp)


def _combine1_body(sp_ref, cp_ref, z_ref, bl_ref, pw_ref, wl_ref, wr_ref,
                   y_ref, zn_ref, ns_ref, cc_ref):
    s = sp_ref[0:_PR, :] + sp_ref[_PR:2 * _PR, :]
    cnt = cp_ref[0:_PR, :] + cp_ref[_PR:2 * _PR, :]
    cc = jnp.maximum(cnt, 1.0)
    h = jax.nn.relu(s / cc + bl_ref[...] + z_ref[...])
    score = jnp.dot(h, pw_ref[...], preferred_element_type=_f32)
    hg = h * jnp.tanh(score)
    y_ref[...] = jnp.dot(hg, wl_ref[...], preferred_element_type=_f32)
    zn_ref[...] = jnp.dot(hg, wr_ref[...], preferred_element_type=_f32)
    ns_ref[...] = -score
    cc_ref[...] = cc


def _combine2_body(sp_ref, cc_ref, z_ref, bl_ref, pw_ref, wl_ref, wr_ref,
                   y_ref, zn_ref, ns_ref):
    s = sp_ref[0:_PR, :] + sp_ref[_PR:2 * _PR, :]
    h = jax.nn.relu(s / cc_ref[...] + bl_ref[...] + z_ref[...])
    score = jnp.dot(h, pw_ref[...], preferred_element_type=_f32)
    hg = h * jnp.tanh(score)
    y_ref[...] = jnp.dot(hg, wl_ref[...], preferred_element_type=_f32)
    zn_ref[...] = jnp.dot(hg, wr_ref[...], preferred_element_type=_f32)
    ns_ref[...] = -score


def _combine3_body(sp_ref, cc_ref, z_ref, bl_ref, pw_ref, hg_ref, ns_ref):
    s = sp_ref[0:_PR, :] + sp_ref[_PR:2 * _PR, :]
    h = jax.nn.relu(s / cc_ref[...] + bl_ref[...] + z_ref[...])
    score = jnp.dot(h, pw_ref[...], preferred_element_type=_f32)
    hg_ref[...] = h * jnp.tanh(score)
    ns_ref[...] = -score


def _sort_body(k3_ref, k2_ref, k1_ref, perm_ref):
    k3 = k3_ref[...]
    k2 = k2_ref[...]
    k1 = k1_ref[...]
    rows, cols = k3.shape
    row = lax.broadcasted_iota(jnp.int32, (rows, cols), 0)
    col = lax.broadcasted_iota(jnp.int32, (rows, cols), 1)
    idx = row * cols + col

    def shifted(x, d, ax):
        n = x.shape[ax]
        d = d % n
        if ax == 0:
            return jnp.concatenate([x[d:, :], x[:d, :]], axis=0)
        return jnp.concatenate([x[:, d:], x[:, :d]], axis=1)

    for ke in range(1, 15):
        big = 1 << ke
        for je in range(ke - 1, -1, -1):
            d = 1 << je
            if d >= cols:
                ax, sh = 0, d // cols
            else:
                ax, sh = 1, d
            lower = (idx & d) == 0
            asc = (idx & big) == 0
            p3 = jnp.where(lower, shifted(k3, sh, ax), shifted(k3, -sh, ax))
            p2 = jnp.where(lower, shifted(k2, sh, ax), shifted(k2, -sh, ax))
            p1 = jnp.where(lower, shifted(k1, sh, ax), shifted(k1, -sh, ax))
            pi = jnp.where(lower, shifted(idx, sh, ax), shifted(idx, -sh, ax))
            lt = (k3 < p3) | ((k3 == p3) & (
                (k2 < p2) | ((k2 == p2) & (
                    (k1 < p1) | ((k1 == p1) & (idx < pi))))))
            take = lt == (lower == asc)
            k3 = jnp.where(take, k3, p3)
            k2 = jnp.where(take, k2, p2)
            k1 = jnp.where(take, k1, p1)
            idx = jnp.where(take, idx, pi)

    perm_ref[...] = jnp.minimum(idx, _N - 1)


def _sort(k3, k2, k1):
    return pl.pallas_call(
        _sort_body,
        out_shape=jax.ShapeDtypeStruct(k3.shape, jnp.int32),
    )(k3, k2, k1)


def _mlp_body(h_ref, w1_ref, b1_ref, w2_ref, b2_ref, w3_ref, b3_ref, o_ref):
    m = jax.nn.relu(
        jnp.dot(h_ref[...], w1_ref[...], preferred_element_type=_f32)
        + b1_ref[...])
    m = jax.nn.relu(
        jnp.dot(m, w2_ref[...], preferred_element_type=_f32) + b2_ref[...])
    lg = jnp.dot(m, w3_ref[...], preferred_element_type=_f32) + b3_ref[...]
    mx = jnp.max(lg, axis=1, keepdims=True)
    sh = lg - mx
    o_ref[...] = sh - jnp.log(jnp.sum(jnp.exp(sh), axis=1, keepdims=True))


def _mlp(h, w1, b1, w2, b2, w3, b3):
    return pl.pallas_call(
        _mlp_body,
        out_shape=jax.ShapeDtypeStruct((h.shape[0], _C), _f32),
    )(h, w1, b1, w2, b2, w3, b3)


def kernel(x, edge_index, edge_weight, Wl1, bl1, Wr1, pw1, Wl2, bl2, Wr2, pw2,
           Wl3, bl3, Wr3, pw3, W1, b1, W2, b2, W3, b3):
    del edge_weight

    ei3 = edge_index.reshape(2, _E // _CHUNK, _CHUNK)
    xp = x.reshape(_N // 8, 8 * _F)

    eye8 = jnp.eye(8, dtype=_f32)

    def bd(w):
        return jnp.kron(eye8, w)

    def pwbd(pw):
        pwn = pw / jnp.linalg.norm(pw)
        return jnp.kron(eye8, pwn[:, None] * jnp.ones((1, _H), _f32))

    def bl8(b):
        return jnp.tile(b, 8).reshape(1, 8 * _H)

    y1, z1 = _proj(xp, bd(Wl1), bd(Wr1))
    s1p, c1p = _seg_sum_cnt(y1, ei3)
    y2, z2, ns1, cc = pl.pallas_call(
        _combine1_body,
        out_shape=(
            jax.ShapeDtypeStruct((_PR, 8 * _H), _f32),
            jax.ShapeDtypeStruct((_PR, 8 * _H), _f32),
            jax.ShapeDtypeStruct((_PR, 8 * _H), _f32),
            jax.ShapeDtypeStruct((_PR, 8 * _H), _f32),
        ),
    )(s1p, c1p, z1, bl8(bl1), pwbd(pw1), bd(Wl2), bd(Wr2))

    s2p, = _seg_sum(y2, ei3)
    y3, z3, ns2 = pl.pallas_call(
        _combine2_body,
        out_shape=(
            jax.ShapeDtypeStruct((_PR, 8 * _H), _f32),
            jax.ShapeDtypeStruct((_PR, 8 * _H), _f32),
            jax.ShapeDtypeStruct((_PR, 8 * _H), _f32),
        ),
    )(s2p, cc, z2, bl8(bl2), pwbd(pw2), bd(Wl3), bd(Wr3))

    s3p, = _seg_sum(y3, ei3)
    hg3, ns3 = pl.pallas_call(
        _combine3_body,
        out_shape=(
            jax.ShapeDtypeStruct((_PR, 8 * _H), _f32),
            jax.ShapeDtypeStruct((_PR, 8 * _H), _f32),
        ),
    )(s3p, cc, z3, bl8(bl3), pwbd(pw3))

    def pad_key(nsb):
        col = nsb[:_N // 8].reshape(_N // 8, 8, _H)[:, :, 0].reshape(_N)
        return jnp.pad(col, (0, _SORT_N - _N),
                       constant_values=_NEG_PAD).reshape(128, 128)

    perm2d = _sort(pad_key(ns3), pad_key(ns2), pad_key(ns1))
    perm = perm2d.reshape(_SORT_N)[:_GOUT].reshape(_GCH, _CHUNK)

    hperm = _gather_rows(hg3, perm)

    return _mlp(hperm[:_N], W1, b1.reshape(1, _H), W2, b2.reshape(1, 8),
                W3, b3.reshape(1, _C))


# speedup vs baseline: 110.7609x; 1.0194x over previous
import functools

import jax
import jax.numpy as jnp
from jax import lax
from jax.experimental import pallas as pl
from jax.experimental.pallas import tpu as pltpu
from jax.experimental.pallas import tpu_sc as plsc

_N = 10000
_E = 320000
_F = 128
_H = 16
_C = 10

_CHUNK = 128
_NC, _NS = 2, 16
_EPAD = 324608
_PAD_DST = 10008
_CORE_CHUNKS = 1252
_Q = 80
_QL = 52
_MAXC = 80
_NP = 10240
_PR = _NP // 8
_PRT = _PR // _NS
_RPT = _NP // _NS
_GCH = 96
_GOUT = _GCH * _CHUNK
_SORT_N = 16384
_NEG_PAD = float("inf")

_f32 = jnp.float32


def _seg_sum_body(with_count, *refs):
    if with_count:
        (y_hbm, ei_hbm, s_out, c_out,
         y_sp, s_sp, c_sp, src_v, dst_v, rows_v, zb_v, b128_v, ones_v,
         gsem, ssem, csem) = refs
    else:
        (y_hbm, ei_hbm, s_out,
         y_sp, s_sp, src_v, dst_v, rows_v, zb_v, b128_v,
         gsem, ssem) = refs
        c_out = c_sp = ones_v = csem = None

    c = lax.axis_index("c")
    s = lax.axis_index("s")
    row0 = s * _RPT
    prow0 = s * _PRT

    def zfill(i, _):
        zb_v[i] = jnp.zeros((_H,), _f32)
        return 0
    lax.fori_loop(0, _RPT, zfill, 0)
    pltpu.sync_copy(zb_v, s_sp.at[pl.ds(row0, _RPT)])
    if with_count:
        pltpu.sync_copy(zb_v, c_sp.at[pl.ds(row0, _RPT)])

        def ofill(i, _):
            ones_v[i] = jnp.ones((_H,), _f32)
            return 0
        lax.fori_loop(0, _CHUNK, ofill, 0)

    pltpu.sync_copy(y_hbm.at[pl.ds(prow0, _PRT)], b128_v)

    def unpackrow(r, _):
        for k in range(8):
            zb_v[r * 8 + k] = b128_v[r, pl.ds(k * _H, _H)]
        return 0
    lax.fori_loop(0, _PRT, unpackrow, 0)
    pltpu.sync_copy(zb_v, y_sp.at[pl.ds(row0, _RPT)])

    nch = jnp.where(s < _NS - 1, _Q, _QL)
    cb = c * _CORE_CHUNKS + s * _Q
    pltpu.sync_copy(ei_hbm.at[0, pl.ds(cb, _MAXC)], src_v)
    pltpu.sync_copy(ei_hbm.at[1, pl.ds(cb, _MAXC)], dst_v)

    plsc.subcore_barrier()

    def gather(j, b):
        return pltpu.async_copy(y_sp.at[src_v.at[j]], rows_v.at[b], gsem.at[b])

    def drain_gather(j, b):
        pltpu.make_async_copy(y_sp.at[src_v.at[j]], rows_v.at[b], gsem.at[b]).wait()

    def scatter(j, b):
        pltpu.async_copy(rows_v.at[b], s_sp.at[dst_v.at[j]], ssem.at[b], add=True)
        if with_count:
            pltpu.async_copy(ones_v, c_sp.at[dst_v.at[j]], csem.at[b], add=True)

    def drain_scatter(j, b):
        pltpu.make_async_copy(rows_v.at[b], s_sp.at[dst_v.at[j]],
                              ssem.at[b]).wait()
        if with_count:
            pltpu.make_async_copy(ones_v, c_sp.at[dst_v.at[j]], csem.at[b]).wait()

    for b in range(4):
        gather(b, b)

    def group(kk, _):
        j0 = 4 * kk
        for b in range(4):
            drain_gather(j0 + b, b)
            scatter(j0 + b, b)
        for b in range(4):
            drain_scatter(j0 + b, b)
            gather(j0 + 4 + b, b)
        return 0
    lax.fori_loop(0, nch // 4 - 1, group, 0)

    jl = nch - 4
    for b in range(4):
        drain_gather(jl + b, b)
        scatter(jl + b, b)
    for b in range(4):
        drain_scatter(jl + b, b)

    plsc.subcore_barrier()

    def packrow(r, _):
        for k in range(8):
            b128_v[r, pl.ds(k * _H, _H)] = zb_v[r * 8 + k]
        return 0

    out_prow = c * _PR + prow0
    pltpu.sync_copy(s_sp.at[pl.ds(row0, _RPT)], zb_v)
    lax.fori_loop(0, _PRT, packrow, 0)
    pltpu.sync_copy(b128_v, s_out.at[pl.ds(out_prow, _PRT)])
    if with_count:
        pltpu.sync_copy(c_sp.at[pl.ds(row0, _RPT)], zb_v)
        lax.fori_loop(0, _PRT, packrow, 0)
        pltpu.sync_copy(b128_v, c_out.at[pl.ds(out_prow, _PRT)])


@functools.lru_cache(maxsize=None)
def _make_seg_sum(with_count):
    mesh = plsc.VectorSubcoreMesh(core_axis_name="c", subcore_axis_name="s", num_cores=_NC, num_subcores=_NS)
    outs = [jax.ShapeDtypeStruct((_NC * _PR, 8 * _H), _f32)]
    scratch = [
        pltpu.VMEM_SHARED((_NP, _H), _f32),
        pltpu.VMEM_SHARED((_NP, _H), _f32),
    ]
    if with_count:
        outs.append(jax.ShapeDtypeStruct((_NC * _PR, 8 * _H), _f32))
        scratch.append(pltpu.VMEM_SHARED((_NP, _H), _f32))
    scratch += [
        pltpu.VMEM((_MAXC, _CHUNK), jnp.int32),
        pltpu.VMEM((_MAXC, _CHUNK), jnp.int32),
        pltpu.VMEM((4, _CHUNK, _H), _f32),
        pltpu.VMEM((_RPT, _H), _f32),
        pltpu.VMEM((_PRT, 8 * _H), _f32),
    ]
    if with_count:
        scratch.append(pltpu.VMEM((_CHUNK, _H), _f32))
    scratch.append(pltpu.SemaphoreType.DMA((4,)))
    scratch.append(pltpu.SemaphoreType.DMA((4,)))
    if with_count:
        scratch.append(pltpu.SemaphoreType.DMA((4,)))
    return pl.kernel(
        functools.partial(_seg_sum_body, with_count),
        out_type=tuple(outs),
        mesh=mesh,
        scratch_types=tuple(scratch),
        compiler_params=pltpu.CompilerParams(use_tc_tiling_on_sc=False),
    )


def _seg_sum_cnt(y, ei3):
    return _make_seg_sum(True)(y, ei3)


def _seg_sum(y, ei3):
    return _make_seg_sum(False)(y, ei3)


def _gather_body(h_hbm, perm_hbm, out_hbm, h_sp, idx_v, rows_v, b16_v, b128_v,
                 sem0, sem1, sem2):
    c = lax.axis_index("c")
    s = lax.axis_index("s")
    wid = s * _NC + c
    base = wid * 3

    pltpu.sync_copy(h_hbm.at[pl.ds(s * _PRT, _PRT)], b128_v)

    def unpackrow(r, _):
        for k in range(8):
            b16_v[r * 8 + k] = b128_v[r, pl.ds(k * _H, _H)]
        return 0
    lax.fori_loop(0, _PRT, unpackrow, 0)
    pltpu.sync_copy(b16_v, h_sp.at[pl.ds(s * _RPT, _RPT)])

    sems = (sem0, sem1, sem2)
    pltpu.sync_copy(perm_hbm.at[pl.ds(base, 3)], idx_v)
    plsc.subcore_barrier()
    cps = [pltpu.async_copy(h_sp.at[idx_v.at[j]], rows_v.at[j], sems[j])
           for j in range(3)]
    for j in range(3):
        cps[j].wait()
        pltpu.sync_copy(rows_v.at[j], out_hbm.at[pl.ds((base + j) * _CHUNK, _CHUNK)])


@functools.lru_cache(maxsize=None)
def _make_gather_rows():
    return pl.kernel(
        _gather_body,
        out_type=jax.ShapeDtypeStruct((_GOUT, _H), _f32),
        mesh=plsc.VectorSubcoreMesh(core_axis_name="c", subcore_axis_name="s",
                                    num_cores=_NC, num_subcores=_NS),
        scratch_types=(
            pltpu.VMEM_SHARED((_NP, _H), _f32),
            pltpu.VMEM((3, _CHUNK), jnp.int32),
            pltpu.VMEM((3, _CHUNK, _H), _f32),
            pltpu.VMEM((_RPT, _H), _f32),
            pltpu.VMEM((_PRT, 8 * _H), _f32),
            pltpu.SemaphoreType.DMA,
            pltpu.SemaphoreType.DMA,
            pltpu.SemaphoreType.DMA,
        ),
        compiler_params=pltpu.CompilerParams(use_tc_tiling_on_sc=False),
    )


def _gather_rows(h, perm):
    return _make_gather_rows()(h, perm)


def _bd(w, reps):
    br, bc = w.shape
    t = jnp.concatenate([w] * 8, axis=0)
    t = jnp.concatenate([t] * 8, axis=1)
    r = lax.broadcasted_iota(jnp.int32, (8 * br, 8 * bc), 0)
    cx = lax.broadcasted_iota(jnp.int32, (8 * br, 8 * bc), 1)
    del reps
    return jnp.where((r // br) == (cx // bc), t, 0.0)


def _group_score(h, pw):
    pvec = jnp.concatenate([pw] * 8, axis=1)
    r = lax.broadcasted_iota(jnp.int32, (8 * _H, 8 * _H), 0)
    cx = lax.broadcasted_iota(jnp.int32, (8 * _H, 8 * _H), 1)
    ones_bd = jnp.where((r // _H) == (cx // _H), jnp.float32(1.0),
                        jnp.float32(0.0))
    nrm = jnp.sqrt(jnp.sum(pw * pw))
    return jnp.dot(h * pvec, ones_bd, preferred_element_type=_f32) / nrm


def _proj_body(x_ref, wl_ref, wr_ref, y_ref, z_ref):
    xv = x_ref[...]
    zpad = jnp.zeros((_PR - _N // 8, 8 * _H), _f32)
    y = jnp.dot(xv, _bd(wl_ref[...], 8), preferred_element_type=_f32)
    z = jnp.dot(xv, _bd(wr_ref[...], 8), preferred_element_type=_f32)
    y_ref[...] = jnp.concatenate([y, zpad], axis=0)
    z_ref[...] = jnp.concatenate([z, zpad], axis=0)


def _proj(xp, wl, wr):
    return pl.pallas_call(
        _proj_body,
        out_shape=(
            jax.ShapeDtypeStruct((_PR, 8 * _H), _f32),
            jax.ShapeDtypeStruct((_PR, 8 * _H), _f32),
        ),
    )(xp, wl, wr)


def _combine1_body(sp_ref, cp_ref, z_ref, bl_ref, pw_ref, wl_ref, wr_ref,
                   y_ref, zn_ref, ns_ref, cc_ref):
    s = sp_ref[0:_PR, :] + sp_ref[_PR:2 * _PR, :]
    cnt = cp_ref[0:_PR, :] + cp_ref[_PR:2 * _PR, :]
    cc = jnp.maximum(cnt, 1.0)
    blv = jnp.concatenate([bl_ref[...]] * 8, axis=1)
    h = jax.nn.relu(s / cc + blv + z_ref[...])
    score = _group_score(h, pw_ref[...])
    hg = h * jnp.tanh(score)
    y_ref[...] = jnp.dot(hg, _bd(wl_ref[...], 8), preferred_element_type=_f32)
    zn_ref[...] = jnp.dot(hg, _bd(wr_ref[...], 8), preferred_element_type=_f32)
    ns_ref[...] = -score
    cc_ref[...] = cc


def _combine2_body(sp_ref, cc_ref, z_ref, bl_ref, pw_ref, wl_ref, wr_ref,
                   y_ref, zn_ref, ns_ref):
    s = sp_ref[0:_PR, :] + sp_ref[_PR:2 * _PR, :]
    blv = jnp.concatenate([bl_ref[...]] * 8, axis=1)
    h = jax.nn.relu(s / cc_ref[...] + blv + z_ref[...])
    score = _group_score(h, pw_ref[...])
    hg = h * jnp.tanh(score)
    y_ref[...] = jnp.dot(hg, _bd(wl_ref[...], 8), preferred_element_type=_f32)
    zn_ref[...] = jnp.dot(hg, _bd(wr_ref[...], 8), preferred_element_type=_f32)
    ns_ref[...] = -score


def _combine3_body(sp_ref, cc_ref, z_ref, bl_ref, pw_ref, hg_ref, ns_ref):
    s = sp_ref[0:_PR, :] + sp_ref[_PR:2 * _PR, :]
    blv = jnp.concatenate([bl_ref[...]] * 8, axis=1)
    h = jax.nn.relu(s / cc_ref[...] + blv + z_ref[...])
    score = _group_score(h, pw_ref[...])
    hg_ref[...] = h * jnp.tanh(score)
    ns_ref[...] = -score


def _sort_body(k3_ref, k2_ref, k1_ref, perm_ref):
    k3 = k3_ref[...]
    k2 = k2_ref[...]
    k1 = k1_ref[...]
    rows, cols = k3.shape
    row = lax.broadcasted_iota(jnp.int32, (rows, cols), 0)
    col = lax.broadcasted_iota(jnp.int32, (rows, cols), 1)
    idx = row * cols + col

    def shifted(x, d, ax):
        n = x.shape[ax]
        d = d % n
        if ax == 0:
            return jnp.concatenate([x[d:, :], x[:d, :]], axis=0)
        return jnp.concatenate([x[:, d:], x[:, :d]], axis=1)

    for ke in range(1, 15):
        big = 1 << ke
        for je in range(ke - 1, -1, -1):
            d = 1 << je
            if d >= cols:
                ax, sh = 0, d // cols
            else:
                ax, sh = 1, d
            lower = (idx & d) == 0
            asc = (idx & big) == 0
            p3 = jnp.where(lower, shifted(k3, sh, ax), shifted(k3, -sh, ax))
            p2 = jnp.where(lower, shifted(k2, sh, ax), shifted(k2, -sh, ax))
            p1 = jnp.where(lower, shifted(k1, sh, ax), shifted(k1, -sh, ax))
            pi = jnp.where(lower, shifted(idx, sh, ax), shifted(idx, -sh, ax))
            lt = (k3 < p3) | ((k3 == p3) & (
                (k2 < p2) | ((k2 == p2) & (
                    (k1 < p1) | ((k1 == p1) & (idx < pi))))))
            take = lt == (lower == asc)
            k3 = jnp.where(take, k3, p3)
            k2 = jnp.where(take, k2, p2)
            k1 = jnp.where(take, k1, p1)
            idx = jnp.where(take, idx, pi)

    perm_ref[...] = jnp.minimum(idx, _N - 1)


def _sort(k3, k2, k1):
    return pl.pallas_call(
        _sort_body,
        out_shape=jax.ShapeDtypeStruct(k3.shape, jnp.int32),
    )(k3, k2, k1)


def _mlp_body(h_ref, w1_ref, b1_ref, w2_ref, b2_ref, w3_ref, b3_ref, o_ref):
    m = jax.nn.relu(
        jnp.dot(h_ref[0:_N, :], w1_ref[...], preferred_element_type=_f32)
        + b1_ref[...])
    m = jax.nn.relu(
        jnp.dot(m, w2_ref[...], preferred_element_type=_f32) + b2_ref[...])
    lg = jnp.dot(m, w3_ref[...], preferred_element_type=_f32) + b3_ref[...]
    mx = jnp.max(lg, axis=1, keepdims=True)
    sh = lg - mx
    o_ref[...] = sh - jnp.log(jnp.sum(jnp.exp(sh), axis=1, keepdims=True))


def _mlp(h, w1, b1, w2, b2, w3, b3):
    return pl.pallas_call(
        _mlp_body,
        out_shape=jax.ShapeDtypeStruct((_N, _C), _f32),
    )(h, w1, b1, w2, b2, w3, b3)


def kernel(x, edge_index, edge_weight, Wl1, bl1, Wr1, pw1, Wl2, bl2, Wr2, pw2,
           Wl3, bl3, Wr3, pw3, W1, b1, W2, b2, W3, b3):
    del edge_weight

    ei3 = jnp.pad(edge_index, ((0, 0), (0, _EPAD - _E)),
                  constant_values=_PAD_DST).reshape(2, _EPAD // _CHUNK, _CHUNK)
    xp = x.reshape(_N // 8, 8 * _F)

    y1, z1 = _proj(xp, Wl1, Wr1)
    s1p, c1p = _seg_sum_cnt(y1, ei3)
    y2, z2, ns1, cc = pl.pallas_call(
        _combine1_body,
        out_shape=(
            jax.ShapeDtypeStruct((_PR, 8 * _H), _f32),
            jax.ShapeDtypeStruct((_PR, 8 * _H), _f32),
            jax.ShapeDtypeStruct((_PR, 8 * _H), _f32),
            jax.ShapeDtypeStruct((_PR, 8 * _H), _f32),
        ),
    )(s1p, c1p, z1, bl1.reshape(1, _H), pw1.reshape(1, _H), Wl2, Wr2)

    s2p, = _seg_sum(y2, ei3)
    y3, z3, ns2 = pl.pallas_call(
        _combine2_body,
        out_shape=(
            jax.ShapeDtypeStruct((_PR, 8 * _H), _f32),
            jax.ShapeDtypeStruct((_PR, 8 * _H), _f32),
            jax.ShapeDtypeStruct((_PR, 8 * _H), _f32),
        ),
    )(s2p, cc, z2, bl2.reshape(1, _H), pw2.reshape(1, _H), Wl3, Wr3)

    s3p, = _seg_sum(y3, ei3)
    hg3, ns3 = pl.pallas_call(
        _combine3_body,
        out_shape=(
            jax.ShapeDtypeStruct((_PR, 8 * _H), _f32),
            jax.ShapeDtypeStruct((_PR, 8 * _H), _f32),
        ),
    )(s3p, cc, z3, bl3.reshape(1, _H), pw3.reshape(1, _H))

    def pad_key(nsb):
        col = nsb[:_N // 8].reshape(_N // 8, 8, _H)[:, :, 0].reshape(_N)
        return jnp.pad(col, (0, _SORT_N - _N),
                       constant_values=_NEG_PAD).reshape(128, 128)

    perm2d = _sort(pad_key(ns3), pad_key(ns2), pad_key(ns1))
    perm = perm2d.reshape(_SORT_N)[:_GOUT].reshape(_GCH, _CHUNK)

    hperm = _gather_rows(hg3, perm)

    return _mlp(hperm, W1, b1.reshape(1, _H), W2, b2.reshape(1, 8),
                W3, b3.reshape(1, _C))


# speedup vs baseline: 118.9631x; 1.0741x over previous
import functools

import jax
import jax.numpy as jnp
from jax import lax
from jax.experimental import pallas as pl
from jax.experimental.pallas import tpu as pltpu
from jax.experimental.pallas import tpu_sc as plsc

_N = 10000
_E = 320000
_F = 128
_H = 16
_C = 10

_CHUNK = 128
_NC, _NS = 2, 16
_EPAD = 324608
_PAD_DST = 10008
_CORE_CHUNKS = 1252
_Q = 80
_QL = 52
_MAXC = 80
_NP = 10240
_PR = _NP // 8
_PRT = _PR // _NS
_RPT = _NP // _NS
_GCH = 96
_GOUT = _GCH * _CHUNK
_SORT_N = 16384
_NEG_PAD = float("inf")

_f32 = jnp.float32


def _seg_sum_body(with_count, *refs):
    if with_count:
        (y_hbm, ei_hbm, s_out, c_out,
         y_sp, s_sp, c_sp, src_v, dst_v, rows_v, zb_v, b128_v, ones_v,
         gsem) = refs
    else:
        (y_hbm, ei_hbm, s_out,
         y_sp, s_sp, src_v, dst_v, rows_v, zb_v, b128_v,
         gsem) = refs
        c_out = c_sp = ones_v = None

    c = lax.axis_index("c")
    s = lax.axis_index("s")
    row0 = s * _RPT
    prow0 = s * _PRT

    def zfill(i, _):
        zb_v[i] = jnp.zeros((_H,), _f32)
        return 0
    lax.fori_loop(0, _RPT, zfill, 0)
    pltpu.sync_copy(zb_v, s_sp.at[pl.ds(row0, _RPT)])
    if with_count:
        pltpu.sync_copy(zb_v, c_sp.at[pl.ds(row0, _RPT)])

        def ofill(i, _):
            ones_v[i] = jnp.ones((_H,), _f32)
            return 0
        lax.fori_loop(0, _CHUNK, ofill, 0)

    pltpu.sync_copy(y_hbm.at[pl.ds(prow0, _PRT)], b128_v)

    def unpackrow(r, _):
        for k in range(8):
            zb_v[r * 8 + k] = b128_v[r, pl.ds(k * _H, _H)]
        return 0
    lax.fori_loop(0, _PRT, unpackrow, 0)
    pltpu.sync_copy(zb_v, y_sp.at[pl.ds(row0, _RPT)])

    nch = jnp.where(s < _NS - 1, _Q, _QL)
    cb = c * _CORE_CHUNKS + s * _Q
    pltpu.sync_copy(ei_hbm.at[0, pl.ds(cb, _MAXC)], src_v)
    pltpu.sync_copy(ei_hbm.at[1, pl.ds(cb, _MAXC)], dst_v)

    plsc.subcore_barrier()

    def gather(j, b):
        return pltpu.async_copy(y_sp.at[src_v.at[j]], rows_v.at[b], gsem.at[b])

    def drain_gather(j, b):
        pltpu.make_async_copy(y_sp.at[src_v.at[j]], rows_v.at[b],
                              gsem.at[b]).wait()

    def scatter(j, b):
        pltpu.sync_copy(rows_v.at[b], s_sp.at[dst_v.at[j]], add=True)
        if with_count:
            pltpu.sync_copy(ones_v, c_sp.at[dst_v.at[j]], add=True)

    gather(0, 0)
    gather(1, 1)

    def pair(jj, _):
        j0 = 2 * jj
        drain_gather(j0, 0)
        scatter(j0, 0)
        gather(j0 + 2, 0)
        drain_gather(j0 + 1, 1)
        scatter(j0 + 1, 1)
        gather(j0 + 3, 1)
        return 0
    lax.fori_loop(0, nch // 2 - 1, pair, 0)

    jl = nch - 2
    drain_gather(jl, 0)
    scatter(jl, 0)
    drain_gather(jl + 1, 1)
    scatter(jl + 1, 1)

    plsc.subcore_barrier()

    def packrow(r, _):
        for k in range(8):
            b128_v[r, pl.ds(k * _H, _H)] = zb_v[r * 8 + k]
        return 0

    out_prow = c * _PR + prow0
    pltpu.sync_copy(s_sp.at[pl.ds(row0, _RPT)], zb_v)
    lax.fori_loop(0, _PRT, packrow, 0)
    pltpu.sync_copy(b128_v, s_out.at[pl.ds(out_prow, _PRT)])
    if with_count:
        pltpu.sync_copy(c_sp.at[pl.ds(row0, _RPT)], zb_v)
        lax.fori_loop(0, _PRT, packrow, 0)
        pltpu.sync_copy(b128_v, c_out.at[pl.ds(out_prow, _PRT)])


@functools.lru_cache(maxsize=None)
def _make_seg_sum(with_count):
    mesh = plsc.VectorSubcoreMesh(core_axis_name="c", subcore_axis_name="s", num_cores=_NC, num_subcores=_NS)
    outs = [jax.ShapeDtypeStruct((_NC * _PR, 8 * _H), _f32)]
    scratch = [
        pltpu.VMEM_SHARED((_NP, _H), _f32),
        pltpu.VMEM_SHARED((_NP, _H), _f32),
    ]
    if with_count:
        outs.append(jax.ShapeDtypeStruct((_NC * _PR, 8 * _H), _f32))
        scratch.append(pltpu.VMEM_SHARED((_NP, _H), _f32))
    scratch += [
        pltpu.VMEM((_MAXC, _CHUNK), jnp.int32),
        pltpu.VMEM((_MAXC, _CHUNK), jnp.int32),
        pltpu.VMEM((2, _CHUNK, _H), _f32),
        pltpu.VMEM((_RPT, _H), _f32),
        pltpu.VMEM((_PRT, 8 * _H), _f32),
    ]
    if with_count:
        scratch.append(pltpu.VMEM((_CHUNK, _H), _f32))
    scratch.append(pltpu.SemaphoreType.DMA((2,)))
    return pl.kernel(
        functools.partial(_seg_sum_body, with_count),
        out_type=tuple(outs),
        mesh=mesh,
        scratch_types=tuple(scratch),
        compiler_params=pltpu.CompilerParams(use_tc_tiling_on_sc=False),
    )


def _seg_sum_cnt(y, ei3):
    return _make_seg_sum(True)(y, ei3)


def _seg_sum(y, ei3):
    return _make_seg_sum(False)(y, ei3)


def _gather_body(h_hbm, perm_hbm, out_hbm, h_sp, idx_v, rows_v, b16_v, b128_v,
                 sem0, sem1, sem2):
    c = lax.axis_index("c")
    s = lax.axis_index("s")
    wid = s * _NC + c
    base = wid * 3

    pltpu.sync_copy(h_hbm.at[pl.ds(s * _PRT, _PRT)], b128_v)

    def unpackrow(r, _):
        for k in range(8):
            b16_v[r * 8 + k] = b128_v[r, pl.ds(k * _H, _H)]
        return 0
    lax.fori_loop(0, _PRT, unpackrow, 0)
    pltpu.sync_copy(b16_v, h_sp.at[pl.ds(s * _RPT, _RPT)])

    sems = (sem0, sem1, sem2)
    pltpu.sync_copy(perm_hbm.at[pl.ds(base, 3)], idx_v)
    plsc.subcore_barrier()
    cps = [pltpu.async_copy(h_sp.at[idx_v.at[j]], rows_v.at[j], sems[j])
           for j in range(3)]
    for j in range(3):
        cps[j].wait()

        def packrow(r, _):
            for k in range(8):
                b128_v[r, pl.ds(k * _H, _H)] = rows_v[j, r * 8 + k]
            return 0
        lax.fori_loop(0, _CHUNK // 8, packrow, 0)
        pltpu.sync_copy(
            b128_v.at[pl.ds(0, _CHUNK // 8)],
            out_hbm.at[pl.ds((base + j) * (_CHUNK // 8), _CHUNK // 8)])


@functools.lru_cache(maxsize=None)
def _make_gather_rows():
    return pl.kernel(
        _gather_body,
        out_type=jax.ShapeDtypeStruct((_GOUT // 8, 8 * _H), _f32),
        mesh=plsc.VectorSubcoreMesh(core_axis_name="c", subcore_axis_name="s",
                                    num_cores=_NC, num_subcores=_NS),
        scratch_types=(
            pltpu.VMEM_SHARED((_NP, _H), _f32),
            pltpu.VMEM((3, _CHUNK), jnp.int32),
            pltpu.VMEM((3, _CHUNK, _H), _f32),
            pltpu.VMEM((_RPT, _H), _f32),
            pltpu.VMEM((_PRT, 8 * _H), _f32),
            pltpu.SemaphoreType.DMA,
            pltpu.SemaphoreType.DMA,
            pltpu.SemaphoreType.DMA,
        ),
        compiler_params=pltpu.CompilerParams(use_tc_tiling_on_sc=False),
    )


def _gather_rows(h, perm):
    return _make_gather_rows()(h, perm)


def _bd(w, reps):
    br, bc = w.shape
    t = jnp.concatenate([w] * 8, axis=0)
    t = jnp.concatenate([t] * 8, axis=1)
    r = lax.broadcasted_iota(jnp.int32, (8 * br, 8 * bc), 0)
    cx = lax.broadcasted_iota(jnp.int32, (8 * br, 8 * bc), 1)
    del reps
    return jnp.where((r // br) == (cx // bc), t, 0.0)


def _group_score(h, pw):
    pvec = jnp.concatenate([pw] * 8, axis=1)
    r = lax.broadcasted_iota(jnp.int32, (8 * _H, 8 * _H), 0)
    cx = lax.broadcasted_iota(jnp.int32, (8 * _H, 8 * _H), 1)
    ones_bd = jnp.where((r // _H) == (cx // _H), jnp.float32(1.0),
                        jnp.float32(0.0))
    nrm = jnp.sqrt(jnp.sum(pw * pw))
    return jnp.dot(h * pvec, ones_bd, preferred_element_type=_f32) / nrm


def _proj_body(x_ref, wl_ref, wr_ref, y_ref, z_ref):
    xv = x_ref[...]
    zpad = jnp.zeros((_PR - _N // 8, 8 * _H), _f32)
    y = jnp.dot(xv, _bd(wl_ref[...], 8), preferred_element_type=_f32)
    z = jnp.dot(xv, _bd(wr_ref[...], 8), preferred_element_type=_f32)
    y_ref[...] = jnp.concatenate([y, zpad], axis=0)
    z_ref[...] = jnp.concatenate([z, zpad], axis=0)


def _proj(xp, wl, wr):
    return pl.pallas_call(
        _proj_body,
        out_shape=(
            jax.ShapeDtypeStruct((_PR, 8 * _H), _f32),
            jax.ShapeDtypeStruct((_PR, 8 * _H), _f32),
        ),
    )(xp, wl, wr)


def _combine1_body(sp_ref, cp_ref, z_ref, bl_ref, pw_ref, wl_ref, wr_ref,
                   y_ref, zn_ref, ns_ref, cc_ref):
    s = sp_ref[0:_PR, :] + sp_ref[_PR:2 * _PR, :]
    cnt = cp_ref[0:_PR, :] + cp_ref[_PR:2 * _PR, :]
    cc = jnp.maximum(cnt, 1.0)
    blv = jnp.concatenate([bl_ref[...]] * 8, axis=1)
    h = jax.nn.relu(s / cc + blv + z_ref[...])
    score = _group_score(h, pw_ref[...])
    hg = h * jnp.tanh(score)
    y_ref[...] = jnp.dot(hg, _bd(wl_ref[...], 8), preferred_element_type=_f32)
    zn_ref[...] = jnp.dot(hg, _bd(wr_ref[...], 8), preferred_element_type=_f32)
    ns_ref[...] = -score
    cc_ref[...] = cc


def _combine2_body(sp_ref, cc_ref, z_ref, bl_ref, pw_ref, wl_ref, wr_ref,
                   y_ref, zn_ref, ns_ref):
    s = sp_ref[0:_PR, :] + sp_ref[_PR:2 * _PR, :]
    blv = jnp.concatenate([bl_ref[...]] * 8, axis=1)
    h = jax.nn.relu(s / cc_ref[...] + blv + z_ref[...])
    score = _group_score(h, pw_ref[...])
    hg = h * jnp.tanh(score)
    y_ref[...] = jnp.dot(hg, _bd(wl_ref[...], 8), preferred_element_type=_f32)
    zn_ref[...] = jnp.dot(hg, _bd(wr_ref[...], 8), preferred_element_type=_f32)
    ns_ref[...] = -score


def _combine3_body(sp_ref, cc_ref, z_ref, bl_ref, pw_ref, hg_ref, ns_ref):
    s = sp_ref[0:_PR, :] + sp_ref[_PR:2 * _PR, :]
    blv = jnp.concatenate([bl_ref[...]] * 8, axis=1)
    h = jax.nn.relu(s / cc_ref[...] + blv + z_ref[...])
    score = _group_score(h, pw_ref[...])
    hg_ref[...] = h * jnp.tanh(score)
    ns_ref[...] = -score


def _sort_body(k3_ref, k2_ref, k1_ref, perm_ref):
    k3 = k3_ref[...]
    k2 = k2_ref[...]
    k1 = k1_ref[...]
    rows, cols = k3.shape
    row = lax.broadcasted_iota(jnp.int32, (rows, cols), 0)
    col = lax.broadcasted_iota(jnp.int32, (rows, cols), 1)
    idx = row * cols + col

    def shifted(x, d, ax):
        n = x.shape[ax]
        d = d % n
        if ax == 0:
            return jnp.concatenate([x[d:, :], x[:d, :]], axis=0)
        return jnp.concatenate([x[:, d:], x[:, :d]], axis=1)

    for ke in range(1, 15):
        big = 1 << ke
        for je in range(ke - 1, -1, -1):
            d = 1 << je
            if d >= cols:
                ax, sh = 0, d // cols
            else:
                ax, sh = 1, d
            lower = (idx & d) == 0
            asc = (idx & big) == 0
            p3 = jnp.where(lower, shifted(k3, sh, ax), shifted(k3, -sh, ax))
            p2 = jnp.where(lower, shifted(k2, sh, ax), shifted(k2, -sh, ax))
            p1 = jnp.where(lower, shifted(k1, sh, ax), shifted(k1, -sh, ax))
            pi = jnp.where(lower, shifted(idx, sh, ax), shifted(idx, -sh, ax))
            lt = (k3 < p3) | ((k3 == p3) & (
                (k2 < p2) | ((k2 == p2) & (
                    (k1 < p1) | ((k1 == p1) & (idx < pi))))))
            take = lt == (lower == asc)
            k3 = jnp.where(take, k3, p3)
            k2 = jnp.where(take, k2, p2)
            k1 = jnp.where(take, k1, p1)
            idx = jnp.where(take, idx, pi)

    perm_ref[...] = jnp.minimum(idx, _N - 1)


def _sort(k3, k2, k1):
    return pl.pallas_call(
        _sort_body,
        out_shape=jax.ShapeDtypeStruct(k3.shape, jnp.int32),
    )(k3, k2, k1)


def _mlp_body(h_ref, w1_ref, b1_ref, w2_ref, b2_ref, w3_ref, b3_ref, o_ref):
    hp = h_ref[0:_N // 8, :]
    m = jax.nn.relu(
        jnp.dot(hp, _bd(w1_ref[...], 8), preferred_element_type=_f32)
        + jnp.concatenate([b1_ref[...]] * 8, axis=1))
    m = jax.nn.relu(
        jnp.dot(m, _bd(w2_ref[...], 8), preferred_element_type=_f32)
        + jnp.concatenate([b2_ref[...]] * 8, axis=1))
    lg = (jnp.dot(m, _bd(w3_ref[...], 8), preferred_element_type=_f32)
          + jnp.concatenate([b3_ref[...]] * 8, axis=1))
    r = lax.broadcasted_iota(jnp.int32, (8 * _C, 8 * _C), 0)
    cx = lax.broadcasted_iota(jnp.int32, (8 * _C, 8 * _C), 1)
    ones_bd = jnp.where((r // _C) == (cx // _C), jnp.float32(1.0),
                        jnp.float32(0.0))
    se = jnp.dot(jnp.exp(lg), ones_bd, preferred_element_type=_f32)
    o_ref[...] = lg - jnp.log(se)


def _mlp(h, w1, b1, w2, b2, w3, b3):
    return pl.pallas_call(
        _mlp_body,
        out_shape=jax.ShapeDtypeStruct((_N // 8, 8 * _C), _f32),
    )(h, w1, b1, w2, b2, w3, b3)


def kernel(x, edge_index, edge_weight, Wl1, bl1, Wr1, pw1, Wl2, bl2, Wr2, pw2,
           Wl3, bl3, Wr3, pw3, W1, b1, W2, b2, W3, b3):
    del edge_weight

    ei3 = jnp.pad(edge_index, ((0, 0), (0, _EPAD - _E)),
                  constant_values=_PAD_DST).reshape(2, _EPAD // _CHUNK, _CHUNK)
    xp = x.reshape(_N // 8, 8 * _F)

    y1, z1 = _proj(xp, Wl1, Wr1)
    s1p, c1p = _seg_sum_cnt(y1, ei3)
    y2, z2, ns1, cc = pl.pallas_call(
        _combine1_body,
        out_shape=(
            jax.ShapeDtypeStruct((_PR, 8 * _H), _f32),
            jax.ShapeDtypeStruct((_PR, 8 * _H), _f32),
            jax.ShapeDtypeStruct((_PR, 8 * _H), _f32),
            jax.ShapeDtypeStruct((_PR, 8 * _H), _f32),
        ),
    )(s1p, c1p, z1, bl1.reshape(1, _H), pw1.reshape(1, _H), Wl2, Wr2)

    s2p, = _seg_sum(y2, ei3)
    y3, z3, ns2 = pl.pallas_call(
        _combine2_body,
        out_shape=(
            jax.ShapeDtypeStruct((_PR, 8 * _H), _f32),
            jax.ShapeDtypeStruct((_PR, 8 * _H), _f32),
            jax.ShapeDtypeStruct((_PR, 8 * _H), _f32),
        ),
    )(s2p, cc, z2, bl2.reshape(1, _H), pw2.reshape(1, _H), Wl3, Wr3)

    s3p, = _seg_sum(y3, ei3)
    hg3, ns3 = pl.pallas_call(
        _combine3_body,
        out_shape=(
            jax.ShapeDtypeStruct((_PR, 8 * _H), _f32),
            jax.ShapeDtypeStruct((_PR, 8 * _H), _f32),
        ),
    )(s3p, cc, z3, bl3.reshape(1, _H), pw3.reshape(1, _H))

    def pad_key(nsb):
        col = nsb[:_N // 8].reshape(_N // 8, 8, _H)[:, :, 0].reshape(_N)
        return jnp.pad(col, (0, _SORT_N - _N),
                       constant_values=_NEG_PAD).reshape(128, 128)

    perm2d = _sort(pad_key(ns3), pad_key(ns2), pad_key(ns1))
    perm = perm2d.reshape(_SORT_N)[:_GOUT].reshape(_GCH, _CHUNK)

    hperm = _gather_rows(hg3, perm)

    outp = _mlp(hperm, W1, b1.reshape(1, _H), W2, b2.reshape(1, 8),
                W3, b3.reshape(1, _C))
    return outp.reshape(_N, _C)
